# Initial kernel scaffold; baseline (speedup 1.0000x reference)
import jax
import jax.numpy as jnp
from jax.experimental import pallas as pl


def kernel(inputs, duty_cycle):
    raise NotImplementedError("write your pallas kernel here")


# speedup vs baseline: 29.5706x; 29.5706x over previous
import jax
import jax.numpy as jnp
from jax.experimental import pallas as pl
from jax.experimental.pallas import tpu as pltpu

_K = 512
_BETA = 1.0


def _kwinner_kernel(x_ref, dc_ref, out_ref):
    x = x_ref[...]
    dc = dc_ref[...]
    units = x.shape[-1]
    target = jnp.float32(_K / units)
    boost = jnp.exp(_BETA * (target - dc))
    boosted = x * boost

    bits = jax.lax.bitcast_convert_type(boosted, jnp.int32)
    key = bits ^ ((bits >> 31) & jnp.int32(0x7FFFFFFF))

    k = jnp.int32(_K)
    cnt_pos = jnp.sum((key >= 0).astype(jnp.int32), axis=1, keepdims=True)
    t0 = jnp.where(cnt_pos >= k, jnp.int32(0), jnp.int32(-2147483648))

    def body(i, t):
        b = jnp.int32(30) - i
        cand = t | (jnp.int32(1) << b)
        cnt = jnp.sum((key >= cand).astype(jnp.int32), axis=1, keepdims=True)
        return jnp.where(cnt >= k, cand, t)

    t = jax.lax.fori_loop(0, 31, body, t0)

    out_ref[...] = jnp.where(key >= t, x, jnp.float32(0.0))


def kernel(inputs, duty_cycle):
    b, f = inputs.shape
    dc2 = duty_cycle.reshape(1, f)
    return pl.pallas_call(
        _kwinner_kernel,
        out_shape=jax.ShapeDtypeStruct((b, f), jnp.float32),
    )(inputs, dc2)


# speedup vs baseline: 30.4438x; 1.0295x over previous
import jax
import jax.numpy as jnp
from jax.experimental import pallas as pl
from jax.experimental.pallas import tpu as pltpu

_K = 512
_BETA = 1.0


def _to_f32(c):
    bits = c ^ ((c >> 31) & jnp.int32(0x7FFFFFFF))
    return jax.lax.bitcast_convert_type(bits, jnp.float32)


def _kwinner_kernel(x_ref, dc_ref, out_ref):
    x = x_ref[...]
    dc = dc_ref[...]
    units = x.shape[-1]
    target = jnp.float32(_K / units)
    boost = jnp.exp(_BETA * (target - dc))
    boosted = x * boost

    k = jnp.float32(_K)

    def count_ge(cand_f):
        flags = jnp.where(boosted >= cand_f, jnp.float32(1.0), jnp.float32(0.0))
        return jnp.sum(flags, axis=1, keepdims=True)

    cnt_pos = count_ge(jnp.float32(0.0))
    t0 = jnp.where(cnt_pos >= k, jnp.int32(0), jnp.int32(-2147483648))

    def body(i, t):
        b = jnp.int32(30) - i
        cand = t | (jnp.int32(1) << b)
        cnt = count_ge(_to_f32(cand))
        return jnp.where(cnt >= k, cand, t)

    t = jax.lax.fori_loop(0, 31, body, t0)
    t_f = _to_f32(t)

    out_ref[...] = jnp.where(boosted >= t_f, x, jnp.float32(0.0))


def kernel(inputs, duty_cycle):
    b, f = inputs.shape
    dc2 = duty_cycle.reshape(1, f)
    return pl.pallas_call(
        _kwinner_kernel,
        out_shape=jax.ShapeDtypeStruct((b, f), jnp.float32),
    )(inputs, dc2)
